# baseline (speedup 1.0000x reference)
import numpy as np
import jax
import jax.numpy as jnp
from jax.experimental import pallas as pl
from jax.experimental.pallas import tpu as pltpu

_B = 128
_V = 100000
_TILE = 8192
_GRID = (_V + _TILE - 1) // _TILE

_K1 = np.uint32(0)
_K2 = np.uint32(42)
_KS2 = np.uint32(0x1BD11BDA) ^ _K1 ^ _K2
_R0 = (13, 15, 26, 6)
_R1 = (17, 29, 16, 24)
_TINY = np.float32(np.finfo(np.float32).tiny)
_NEG_INF = np.float32(-np.inf)
_BIG = np.int32(np.iinfo(np.int32).max)


def _rotl(x, r):
    return (x << np.uint32(r)) | (x >> np.uint32(32 - r))


def _rounds(x0, x1, rots):
    for r in rots:
        x0 = x0 + x1
        x1 = _rotl(x1, r) ^ x0
    return x0, x1


def _sample_kernel(lp_ref, out_ref, best_val, best_idx):
    j = pl.program_id(0)
    shape = lp_ref.shape

    b = jax.lax.broadcasted_iota(jnp.uint32, shape, 0)
    vglob = jax.lax.broadcasted_iota(jnp.int32, shape, 1) + j * _TILE

    x1 = b * np.uint32(_V) + vglob.astype(jnp.uint32) + _K2
    x0 = x1
    x1 = _rotl(x1, _R0[0]) ^ x0
    x0, x1 = _rounds(x0, x1, _R0[1:])
    x0, x1 = x0 + _K2, x1 + (_KS2 + np.uint32(1))
    x0, x1 = _rounds(x0, x1, _R1)
    x0, x1 = x0 + _KS2, x1 + (_K1 + np.uint32(2))
    x0, x1 = _rounds(x0, x1, _R0)
    x0, x1 = x0 + _K1, x1 + (_K2 + np.uint32(3))
    x0, x1 = _rounds(x0, x1, _R1)
    x0, x1 = x0 + _K2, x1 + (_KS2 + np.uint32(4))
    x0, x1 = _rounds(x0, x1, _R0)
    bits = (x0 + _KS2) ^ (x1 + (_K1 + np.uint32(5)))

    fb = (bits >> np.uint32(9)) | np.uint32(0x3F800000)
    u = jnp.maximum(
        jax.lax.bitcast_convert_type(fb, jnp.float32) - np.float32(1.0), _TINY)
    g = -jnp.log(-jnp.log(u))

    t = g + lp_ref[...]
    t = jnp.where(vglob < _V, t, _NEG_INF)

    m = jnp.max(t, axis=1, keepdims=True)
    a = jnp.min(jnp.where(t == m, vglob, _BIG), axis=1, keepdims=True)

    @pl.when(j == 0)
    def _():
        best_val[...] = m
        best_idx[...] = a

    @pl.when(j != 0)
    def _():
        upd = m > best_val[...]
        best_val[...] = jnp.where(upd, m, best_val[...])
        best_idx[...] = jnp.where(upd, a, best_idx[...])

    @pl.when(j == _GRID - 1)
    def _():
        out_ref[...] = best_idx[...]


def kernel(log_p):
    out = pl.pallas_call(
        _sample_kernel,
        grid=(_GRID,),
        in_specs=[pl.BlockSpec((_B, _TILE), lambda j: (0, j))],
        out_specs=pl.BlockSpec((_B, 1), lambda j: (0, 0)),
        out_shape=jax.ShapeDtypeStruct((_B, 1), jnp.int32),
        scratch_shapes=[
            pltpu.VMEM((_B, 1), jnp.float32),
            pltpu.VMEM((_B, 1), jnp.int32),
        ],
        compiler_params=pltpu.CompilerParams(
            dimension_semantics=("arbitrary",)),
    )(log_p)
    return out.reshape(_B)


# speedup vs baseline: 1.0647x; 1.0647x over previous
import numpy as np
import jax
import jax.numpy as jnp
from jax.experimental import pallas as pl
from jax.experimental.pallas import tpu as pltpu

_B = 128
_V = 100000
_TILE = 2000
_GRID = _V // _TILE

_K1 = np.uint32(0)
_K2 = np.uint32(42)
_KS2 = np.uint32(0x1BD11BDA) ^ _K1 ^ _K2
_R0 = (13, 15, 26, 6)
_R1 = (17, 29, 16, 24)
_TINY = np.float32(np.finfo(np.float32).tiny)
_BIG = np.int32(np.iinfo(np.int32).max)


def _rotl(x, r):
    return (x << np.uint32(r)) | (x >> np.uint32(32 - r))


def _rounds(x0, x1, rots):
    for r in rots:
        x0 = x0 + x1
        x1 = _rotl(x1, r) ^ x0
    return x0, x1


def _sample_kernel(lp_ref, out_ref, best_val, best_idx):
    j = pl.program_id(0)
    shape = lp_ref.shape

    lane_b = jax.lax.broadcasted_iota(jnp.uint32, shape, 1)
    vj = jax.lax.broadcasted_iota(jnp.int32, shape, 0) + j * _TILE

    x1 = (lane_b * np.uint32(_V) + _K2) + vj.astype(jnp.uint32)
    x0 = x1
    x1 = _rotl(x1, _R0[0]) ^ x0
    x0, x1 = _rounds(x0, x1, _R0[1:])
    x0, x1 = x0 + _K2, x1 + (_KS2 + np.uint32(1))
    x0, x1 = _rounds(x0, x1, _R1)
    x0, x1 = x0 + _KS2, x1 + (_K1 + np.uint32(2))
    x0, x1 = _rounds(x0, x1, _R0)
    x0, x1 = x0 + _K1, x1 + (_K2 + np.uint32(3))
    x0, x1 = _rounds(x0, x1, _R1)
    x0, x1 = x0 + _K2, x1 + (_KS2 + np.uint32(4))
    x0, x1 = _rounds(x0, x1, _R0)
    bits = (x0 + _KS2) ^ (x1 + (_K1 + np.uint32(5)))

    fb = (bits >> np.uint32(9)) | np.uint32(0x3F800000)
    u = jnp.maximum(
        jax.lax.bitcast_convert_type(fb, jnp.float32) - np.float32(1.0), _TINY)
    g = -jnp.log(-jnp.log(u))

    t = g + lp_ref[...]

    m = jnp.max(t, axis=0, keepdims=True)
    a = jnp.min(jnp.where(t == m, vj, _BIG), axis=0, keepdims=True)

    @pl.when(j == 0)
    def _():
        best_val[...] = m
        best_idx[...] = a

    @pl.when(j != 0)
    def _():
        upd = m > best_val[...]
        best_val[...] = jnp.where(upd, m, best_val[...])
        best_idx[...] = jnp.where(upd, a, best_idx[...])

    @pl.when(j == _GRID - 1)
    def _():
        out_ref[...] = best_idx[...]


def kernel(log_p):
    out = pl.pallas_call(
        _sample_kernel,
        grid=(_GRID,),
        in_specs=[pl.BlockSpec((_TILE, _B), lambda j: (j, 0))],
        out_specs=pl.BlockSpec((1, _B), lambda j: (0, 0)),
        out_shape=jax.ShapeDtypeStruct((1, _B), jnp.int32),
        scratch_shapes=[
            pltpu.VMEM((1, _B), jnp.float32),
            pltpu.VMEM((1, _B), jnp.int32),
        ],
        compiler_params=pltpu.CompilerParams(
            dimension_semantics=("arbitrary",)),
    )(log_p.T)
    return out.reshape(_B)


# speedup vs baseline: 1.7045x; 1.6010x over previous
import numpy as np
import jax
import jax.numpy as jnp
from jax.experimental import pallas as pl
from jax.experimental.pallas import tpu as pltpu

_B = 128
_V = 100000
_TILE = 2048
_GRID = (_V + _TILE - 1) // _TILE

_K1 = np.uint32(0)
_K2 = np.uint32(42)
_KS2 = np.uint32(0x1BD11BDA) ^ _K1 ^ _K2
_R0 = (13, 15, 26, 6)
_R1 = (17, 29, 16, 24)
_TINY = np.float32(np.finfo(np.float32).tiny)
_NEG_INF = np.float32(-np.inf)
_BIG = np.int32(np.iinfo(np.int32).max)


def _rotl(x, r):
    return (x << np.uint32(r)) | (x >> np.uint32(32 - r))


def _rounds(x0, x1, rots):
    for r in rots:
        x0 = x0 + x1
        x1 = _rotl(x1, r) ^ x0
    return x0, x1


def _sample_kernel(lp_ref, out_ref, best_val, best_idx):
    j = pl.program_id(0)
    shape = (_B, _TILE)

    b = jax.lax.broadcasted_iota(jnp.uint32, shape, 0)
    vglob = jax.lax.broadcasted_iota(jnp.int32, shape, 1) + j * _TILE

    x1 = (b * np.uint32(_V) + _K2) + vglob.astype(jnp.uint32)
    x0 = x1
    x1 = _rotl(x1, _R0[0]) ^ x0
    x0, x1 = _rounds(x0, x1, _R0[1:])
    x0, x1 = x0 + _K2, x1 + (_KS2 + np.uint32(1))
    x0, x1 = _rounds(x0, x1, _R1)
    x0, x1 = x0 + _KS2, x1 + (_K1 + np.uint32(2))
    x0, x1 = _rounds(x0, x1, _R0)
    x0, x1 = x0 + _K1, x1 + (_K2 + np.uint32(3))
    x0, x1 = _rounds(x0, x1, _R1)
    x0, x1 = x0 + _K2, x1 + (_KS2 + np.uint32(4))
    x0, x1 = _rounds(x0, x1, _R0)
    bits = (x0 + _KS2) ^ (x1 + (_K1 + np.uint32(5)))

    fb = (bits >> np.uint32(9)) | np.uint32(0x3F800000)
    u = jnp.maximum(
        jax.lax.bitcast_convert_type(fb, jnp.float32) - np.float32(1.0), _TINY)
    g = -jnp.log(-jnp.log(u))

    t = g + jnp.transpose(lp_ref[...])
    t = jnp.where(vglob < _V, t, _NEG_INF)

    m = jnp.max(t, axis=1, keepdims=True)
    a = jnp.min(jnp.where(t == m, vglob, _BIG), axis=1, keepdims=True)

    @pl.when(j == 0)
    def _():
        best_val[...] = m
        best_idx[...] = a

    @pl.when(j != 0)
    def _():
        upd = m > best_val[...]
        best_val[...] = jnp.where(upd, m, best_val[...])
        best_idx[...] = jnp.where(upd, a, best_idx[...])

    @pl.when(j == _GRID - 1)
    def _():
        out_ref[...] = best_idx[...]


def kernel(log_p):
    out = pl.pallas_call(
        _sample_kernel,
        grid=(_GRID,),
        in_specs=[pl.BlockSpec((_TILE, _B), lambda j: (j, 0))],
        out_specs=pl.BlockSpec((_B, 1), lambda j: (0, 0)),
        out_shape=jax.ShapeDtypeStruct((_B, 1), jnp.int32),
        scratch_shapes=[
            pltpu.VMEM((_B, 1), jnp.float32),
            pltpu.VMEM((_B, 1), jnp.int32),
        ],
        compiler_params=pltpu.CompilerParams(
            dimension_semantics=("arbitrary",)),
    )(log_p.T)
    return out.reshape(_B)
